# baseline (speedup 1.0000x reference)
import functools

import jax
import jax.numpy as jnp
from jax import lax
from jax.experimental import pallas as pl
from jax.experimental.pallas import tpu as pltpu
from jax.experimental.pallas import tpu_sc as plsc

N = 10000
E = 160000
T = 12
C = 32
D = T * C
NQ = 4
HQ = D // NQ
NP = 10240
NC = 2
NS = 16
KE = 128
NCH = 80
EPW_S = KE * NCH
EP_S = NS * EPW_S
EPW_H = EP_S // 32
JUNK = N + 16
NBUF = 4


def _deg_body(ei_hbm, dego_hbm, degi_hbm,
              ebuf, hist_o, hist_i, rbuf, sh_o, sh_i):
    cid = lax.axis_index("c")
    sid = lax.axis_index("s")
    w = sid * NC + cid
    zero16 = jnp.zeros((16,), jnp.float32)
    one16 = jnp.ones((16,), jnp.float32)
    lane = lax.broadcasted_iota(jnp.int32, (16,), 0)

    def zero_hist(i, _):
        hist_o[pl.ds(i * 16, 16)] = zero16
        hist_i[pl.ds(i * 16, 16)] = zero16
        return 0
    lax.fori_loop(0, NP // 16, zero_hist, 0)

    def do_hist(base, hist):
        pltpu.sync_copy(ei_hbm.at[pl.ds(base + w * EPW_H, EPW_H)], ebuf)

        def acc(j, _):
            ids = ebuf[pl.ds(j * 16, 16)]
            ids = jnp.where(lane + (w * EPW_H + j * 16) >= E, JUNK, ids)
            plsc.addupdate_scatter(hist, [ids], one16)
            return 0
        lax.fori_loop(0, EPW_H // 16, acc, 0)

    do_hist(0, hist_o)
    do_hist(EP_S, hist_i)

    pltpu.sync_copy(hist_o, sh_o.at[sid])
    pltpu.sync_copy(hist_i, sh_i.at[sid])
    plsc.subcore_barrier()

    cols = NP // NS

    def reduce_one(sh, hist, out_hbm):
        pltpu.sync_copy(sh.at[:, pl.ds(sid * cols, cols)], rbuf)

        def red(i, _):
            s = rbuf[0, pl.ds(i * 16, 16)]
            for k in range(1, NS):
                s = s + rbuf[k, pl.ds(i * 16, 16)]
            hist[pl.ds(i * 16, 16)] = s
            return 0
        lax.fori_loop(0, cols // 16, red, 0)
        pltpu.sync_copy(hist.at[pl.ds(0, cols)],
                        out_hbm.at[pl.ds(cid * NP + sid * cols, cols)])

    reduce_one(sh_o, hist_o, dego_hbm)
    reduce_one(sh_i, hist_i, degi_hbm)


def _scat_body(y_hbm, ei_hbm, zeros_hbm, out_hbm,
               sidx, didx, r0, r1, r2, r3,
               g0, g1, g2, g3, s0, s1, s2, s3, acc):
    cid = lax.axis_index("c")
    sid = lax.axis_index("s")
    rows = NP // NS
    rbufs = (r0, r1, r2, r3)
    gsems = (g0, g1, g2, g3)
    ssems = (s0, s1, s2, s3)
    lane = lax.broadcasted_iota(jnp.int32, (16,), 0)

    pltpu.sync_copy(ei_hbm.at[pl.ds(sid * NCH, NCH)], sidx)
    pltpu.sync_copy(ei_hbm.at[pl.ds((NS + sid) * NCH, NCH)], didx)

    @pl.when(sid == NS - 1)
    def _():
        def fix(r, _):
            for i in range(KE // 16):
                e0 = sid * EPW_S + r * KE + i * 16
                v = didx[r, pl.ds(i * 16, 16)]
                didx[r, pl.ds(i * 16, 16)] = jnp.where(
                    lane + e0 >= E, JUNK, v)
            return 0
        lax.fori_loop(0, NCH, fix, 0)

    for p in range(NQ // NC):
        q = cid * (NQ // NC) + p

        def mk(r, _):
            for i in range(KE // 16):
                s16 = sidx[r, pl.ds(i * 16, 16)]
                sidx[r, pl.ds(i * 16, 16)] = (
                    s16 * NQ + q if p == 0 else s16 + 1)
            return 0
        lax.fori_loop(0, NCH, mk, 0)

        pltpu.sync_copy(zeros_hbm, r0)
        for k in range(rows // KE):
            pltpu.sync_copy(r0, acc.at[pl.ds(sid * rows + k * KE, KE)])
        plsc.subcore_barrier()

        for b in range(NBUF):
            pltpu.async_copy(y_hbm.at[sidx.at[b]], rbufs[b], gsems[b])

        def body(jj, _):
            for b in range(NBUF):
                j = jj * NBUF + b
                rb, gs, ss = rbufs[b], gsems[b], ssems[b]
                pltpu.make_async_copy(y_hbm.at[sidx.at[j]], rb, gs).wait()
                pltpu.async_copy(rb, acc.at[didx.at[j]], ss, add=True)

                @pl.when(j < NCH - NBUF)
                def _():
                    pltpu.make_async_copy(rb, acc.at[didx.at[j]], ss).wait()
                    pltpu.async_copy(y_hbm.at[sidx.at[j + NBUF]], rb, gs)
            return 0
        lax.fori_loop(0, NCH // NBUF, body, 0)

        for b in range(NBUF):
            j = NCH - NBUF + b
            pltpu.make_async_copy(rbufs[b], acc.at[didx.at[j]],
                                  ssems[b]).wait()
        plsc.subcore_barrier()

        pltpu.sync_copy(
            acc.at[pl.ds(sid * rows, rows)],
            out_hbm.at[pl.ds(sid * rows, rows), pl.ds(q * HQ, HQ)])
        plsc.subcore_barrier()


@functools.lru_cache(maxsize=None)
def _sc_kernels():
    mesh = plsc.VectorSubcoreMesh(
        core_axis_name="c", subcore_axis_name="s",
        num_cores=NC, num_subcores=NS)
    params = pltpu.CompilerParams(needs_layout_passes=False,
                                  use_tc_tiling_on_sc=False)
    deg_kernel = functools.partial(
        pl.kernel,
        out_type=(jax.ShapeDtypeStruct((NC * NP,), jnp.float32),
                  jax.ShapeDtypeStruct((NC * NP,), jnp.float32)),
        mesh=mesh,
        compiler_params=params,
        scratch_types=[
            pltpu.VMEM((EPW_H,), jnp.int32),
            pltpu.VMEM((NP,), jnp.float32),
            pltpu.VMEM((NP,), jnp.float32),
            pltpu.VMEM((NS, NP // NS), jnp.float32),
            pltpu.VMEM_SHARED((NS, NP), jnp.float32),
            pltpu.VMEM_SHARED((NS, NP), jnp.float32),
        ],
    )(_deg_body)
    scat_kernel = functools.partial(
        pl.kernel,
        out_type=jax.ShapeDtypeStruct((NP, D), jnp.float32),
        mesh=mesh,
        compiler_params=params,
        scratch_types=(
            [pltpu.VMEM((NCH, KE), jnp.int32)] * 2
            + [pltpu.VMEM((KE, HQ), jnp.float32)] * NBUF
            + [pltpu.SemaphoreType.DMA] * (2 * NBUF)
            + [pltpu.VMEM_SHARED((NP, HQ), jnp.float32)]),
    )(_scat_body)
    return deg_kernel, scat_kernel


def _prep_body(x_ref, d_ref, w_ref, o_ref):
    deg = jnp.sum(d_ref[...], axis=1, keepdims=True)
    norm = lax.rsqrt(jnp.maximum(deg, 1.0))
    xb = x_ref[...] * norm
    wm = w_ref[...]
    for t in range(T):
        o_ref[:, t * C:(t + 1) * C] = jnp.dot(
            xb[:, t * C:(t + 1) * C], wm, preferred_element_type=jnp.float32)


def _epi_body(a_ref, d_ref, b_ref, o_ref):
    deg = jnp.sum(d_ref[...], axis=1, keepdims=True)
    norm = lax.rsqrt(jnp.maximum(deg, 1.0))
    ab = a_ref[...] * norm
    bb = b_ref[...]
    for t in range(T):
        o_ref[:, t * C:(t + 1) * C] = jnp.maximum(
            ab[:, t * C:(t + 1) * C] + bb, 0.0)


def kernel(x, edge_index, W, b):
    ei_p = jnp.pad(edge_index, ((0, 0), (0, EP_S - E)))

    deg_kernel, scat_kernel = _sc_kernels()
    dego_f, degi_f = deg_kernel(ei_p.reshape(2 * EP_S))
    dego = dego_f.reshape(NC, NP).T
    degi = degi_f.reshape(NC, NP).T

    x0 = jnp.pad(x[0], ((0, 0), (0, 0), (0, NP - N)))
    xr = x0.transpose(2, 1, 0).reshape(NP, D)

    grid = NP // 128
    y = pl.pallas_call(
        _prep_body,
        grid=(grid,),
        in_specs=[pl.BlockSpec((128, D), lambda i: (i, 0)),
                  pl.BlockSpec((128, NC), lambda i: (i, 0)),
                  pl.BlockSpec((C, C), lambda i: (0, 0))],
        out_specs=pl.BlockSpec((128, D), lambda i: (i, 0)),
        out_shape=jax.ShapeDtypeStruct((NP, D), jnp.float32),
    )(xr, dego, W)

    y_tab = y.reshape(NQ * NP, HQ)
    zeros = jnp.zeros((KE, HQ), jnp.float32)
    agg = scat_kernel(y_tab, ei_p.reshape(2 * NS * NCH, KE), zeros)

    z = pl.pallas_call(
        _epi_body,
        grid=(grid,),
        in_specs=[pl.BlockSpec((128, D), lambda i: (i, 0)),
                  pl.BlockSpec((128, NC), lambda i: (i, 0)),
                  pl.BlockSpec((1, C), lambda i: (0, 0))],
        out_specs=pl.BlockSpec((128, D), lambda i: (i, 0)),
        out_shape=jax.ShapeDtypeStruct((NP, D), jnp.float32),
    )(agg, degi, b[None])

    return z[:N].reshape(N, T, C).transpose(2, 1, 0)[None]


# speedup vs baseline: 1.1777x; 1.1777x over previous
import functools

import jax
import jax.numpy as jnp
from jax import lax
from jax.experimental import pallas as pl
from jax.experimental.pallas import tpu as pltpu
from jax.experimental.pallas import tpu_sc as plsc

N = 10000
E = 160000
T = 12
C = 32
D = T * C
NQ = 2
HQ = D // NQ
NP = 10240
NC = 2
NS = 16
KE = 128
NCH = 80
EPW_S = KE * NCH
EP_S = NS * EPW_S
EPW_H = EP_S // 32
JUNK = N + 16
NBUF = 4


def _deg_body(ei_hbm, dego_hbm, degi_hbm,
              ebuf, hist_o, hist_i, rbuf, sh_o, sh_i):
    cid = lax.axis_index("c")
    sid = lax.axis_index("s")
    w = sid * NC + cid
    zero16 = jnp.zeros((16,), jnp.float32)
    one16 = jnp.ones((16,), jnp.float32)
    lane = lax.broadcasted_iota(jnp.int32, (16,), 0)

    def zero_hist(i, _):
        hist_o[pl.ds(i * 16, 16)] = zero16
        hist_i[pl.ds(i * 16, 16)] = zero16
        return 0
    lax.fori_loop(0, NP // 16, zero_hist, 0)

    def do_hist(base, hist):
        pltpu.sync_copy(ei_hbm.at[pl.ds(base + w * EPW_H, EPW_H)], ebuf)

        def acc(j, _):
            ids = ebuf[pl.ds(j * 16, 16)]
            ids = jnp.where(lane + (w * EPW_H + j * 16) >= E, JUNK, ids)
            plsc.addupdate_scatter(hist, [ids], one16)
            return 0
        lax.fori_loop(0, EPW_H // 16, acc, 0)

    do_hist(0, hist_o)
    do_hist(EP_S, hist_i)

    pltpu.sync_copy(hist_o, sh_o.at[sid])
    pltpu.sync_copy(hist_i, sh_i.at[sid])
    plsc.subcore_barrier()

    cols = NP // NS

    def reduce_one(sh, hist, out_hbm):
        pltpu.sync_copy(sh.at[:, pl.ds(sid * cols, cols)], rbuf)

        def red(i, _):
            s = rbuf[0, pl.ds(i * 16, 16)]
            for k in range(1, NS):
                s = s + rbuf[k, pl.ds(i * 16, 16)]
            hist[pl.ds(i * 16, 16)] = s
            return 0
        lax.fori_loop(0, cols // 16, red, 0)
        pltpu.sync_copy(hist.at[pl.ds(0, cols)],
                        out_hbm.at[pl.ds(cid * NP + sid * cols, cols)])

    reduce_one(sh_o, hist_o, dego_hbm)
    reduce_one(sh_i, hist_i, degi_hbm)


def _scat_body(y_hbm, ei_hbm, zeros_hbm, out_hbm,
               sidx, didx, r0, r1, r2, r3,
               g0, g1, g2, g3, s0, s1, s2, s3, acc):
    cid = lax.axis_index("c")
    sid = lax.axis_index("s")
    rows = NP // NS
    rbufs = (r0, r1, r2, r3)
    gsems = (g0, g1, g2, g3)
    ssems = (s0, s1, s2, s3)
    lane = lax.broadcasted_iota(jnp.int32, (16,), 0)

    pltpu.sync_copy(ei_hbm.at[pl.ds(sid * NCH, NCH)], sidx)
    pltpu.sync_copy(ei_hbm.at[pl.ds((NS + sid) * NCH, NCH)], didx)

    @pl.when(sid == NS - 1)
    def _():
        def fix(r, _):
            for i in range(KE // 16):
                e0 = sid * EPW_S + r * KE + i * 16
                v = didx[r, pl.ds(i * 16, 16)]
                didx[r, pl.ds(i * 16, 16)] = jnp.where(
                    lane + e0 >= E, JUNK, v)
            return 0
        lax.fori_loop(0, NCH, fix, 0)

    for p in range(NQ // NC):
        q = cid * (NQ // NC) + p

        def mk(r, _):
            for i in range(KE // 16):
                s16 = sidx[r, pl.ds(i * 16, 16)]
                sidx[r, pl.ds(i * 16, 16)] = (
                    s16 * NQ + q if p == 0 else s16 + 1)
            return 0
        lax.fori_loop(0, NCH, mk, 0)

        pltpu.sync_copy(zeros_hbm, r0)
        for k in range(rows // KE):
            pltpu.sync_copy(r0, acc.at[pl.ds(sid * rows + k * KE, KE)])
        plsc.subcore_barrier()

        for b in range(NBUF):
            pltpu.async_copy(y_hbm.at[sidx.at[b]], rbufs[b], gsems[b])

        def body(jj, _):
            for b in range(NBUF):
                j = jj * NBUF + b
                rb, gs, ss = rbufs[b], gsems[b], ssems[b]
                pltpu.make_async_copy(y_hbm.at[sidx.at[j]], rb, gs).wait()
                pltpu.async_copy(rb, acc.at[didx.at[j]], ss, add=True)

                @pl.when(j < NCH - NBUF)
                def _():
                    pltpu.make_async_copy(rb, acc.at[didx.at[j]], ss).wait()
                    pltpu.async_copy(y_hbm.at[sidx.at[j + NBUF]], rb, gs)
            return 0
        lax.fori_loop(0, NCH // NBUF, body, 0)

        for b in range(NBUF):
            j = NCH - NBUF + b
            pltpu.make_async_copy(rbufs[b], acc.at[didx.at[j]],
                                  ssems[b]).wait()
        plsc.subcore_barrier()

        pltpu.sync_copy(
            acc.at[pl.ds(sid * rows, rows)],
            out_hbm.at[pl.ds(sid * rows, rows), pl.ds(q * HQ, HQ)])
        plsc.subcore_barrier()


@functools.lru_cache(maxsize=None)
def _sc_kernels():
    mesh = plsc.VectorSubcoreMesh(
        core_axis_name="c", subcore_axis_name="s",
        num_cores=NC, num_subcores=NS)
    params = pltpu.CompilerParams(needs_layout_passes=False,
                                  use_tc_tiling_on_sc=False)
    deg_kernel = functools.partial(
        pl.kernel,
        out_type=(jax.ShapeDtypeStruct((NC * NP,), jnp.float32),
                  jax.ShapeDtypeStruct((NC * NP,), jnp.float32)),
        mesh=mesh,
        compiler_params=params,
        scratch_types=[
            pltpu.VMEM((EPW_H,), jnp.int32),
            pltpu.VMEM((NP,), jnp.float32),
            pltpu.VMEM((NP,), jnp.float32),
            pltpu.VMEM((NS, NP // NS), jnp.float32),
            pltpu.VMEM_SHARED((NS, NP), jnp.float32),
            pltpu.VMEM_SHARED((NS, NP), jnp.float32),
        ],
    )(_deg_body)
    scat_kernel = functools.partial(
        pl.kernel,
        out_type=jax.ShapeDtypeStruct((NP, D), jnp.bfloat16),
        mesh=mesh,
        compiler_params=params,
        scratch_types=(
            [pltpu.VMEM((NCH, KE), jnp.int32)] * 2
            + [pltpu.VMEM((KE, HQ), jnp.bfloat16)] * NBUF
            + [pltpu.SemaphoreType.DMA] * (2 * NBUF)
            + [pltpu.VMEM_SHARED((NP, HQ), jnp.bfloat16)]),
    )(_scat_body)
    return deg_kernel, scat_kernel


def _prep_body(x_ref, d_ref, w_ref, o_ref):
    deg = jnp.sum(d_ref[...], axis=1, keepdims=True)
    norm = lax.rsqrt(jnp.maximum(deg, 1.0))
    xb = x_ref[...] * norm
    wm = w_ref[...]
    for t in range(T):
        o_ref[:, t * C:(t + 1) * C] = jnp.dot(
            xb[:, t * C:(t + 1) * C], wm,
            preferred_element_type=jnp.float32).astype(jnp.bfloat16)


def _epi_body(a_ref, d_ref, b_ref, o_ref):
    deg = jnp.sum(d_ref[...], axis=1, keepdims=True)
    norm = lax.rsqrt(jnp.maximum(deg, 1.0))
    ab = a_ref[...].astype(jnp.float32) * norm
    bb = b_ref[...]
    for t in range(T):
        o_ref[:, t * C:(t + 1) * C] = jnp.maximum(
            ab[:, t * C:(t + 1) * C] + bb, 0.0)


def kernel(x, edge_index, W, b):
    ei_p = jnp.pad(edge_index, ((0, 0), (0, EP_S - E)))

    deg_kernel, scat_kernel = _sc_kernels()
    dego_f, degi_f = deg_kernel(ei_p.reshape(2 * EP_S))
    dego = dego_f.reshape(NC, NP).T
    degi = degi_f.reshape(NC, NP).T

    x0 = jnp.pad(x[0], ((0, 0), (0, 0), (0, NP - N)))
    xr = x0.transpose(2, 1, 0).reshape(NP, D)

    grid = NP // 128
    y = pl.pallas_call(
        _prep_body,
        grid=(grid,),
        in_specs=[pl.BlockSpec((128, D), lambda i: (i, 0)),
                  pl.BlockSpec((128, NC), lambda i: (i, 0)),
                  pl.BlockSpec((C, C), lambda i: (0, 0))],
        out_specs=pl.BlockSpec((128, D), lambda i: (i, 0)),
        out_shape=jax.ShapeDtypeStruct((NP, D), jnp.bfloat16),
    )(xr, dego, W)

    y_tab = y.reshape(NQ * NP, HQ)
    zeros = jnp.zeros((KE, HQ), jnp.bfloat16)
    agg = scat_kernel(y_tab, ei_p.reshape(2 * NS * NCH, KE), zeros)

    z = pl.pallas_call(
        _epi_body,
        grid=(grid,),
        in_specs=[pl.BlockSpec((128, D), lambda i: (i, 0)),
                  pl.BlockSpec((128, NC), lambda i: (i, 0)),
                  pl.BlockSpec((1, C), lambda i: (0, 0))],
        out_specs=pl.BlockSpec((128, D), lambda i: (i, 0)),
        out_shape=jax.ShapeDtypeStruct((NP, D), jnp.float32),
    )(agg, degi, b[None])

    return z[:N].reshape(N, T, C).transpose(2, 1, 0)[None]


# speedup vs baseline: 1.1811x; 1.0029x over previous
import functools

import jax
import jax.numpy as jnp
from jax import lax
from jax.experimental import pallas as pl
from jax.experimental.pallas import tpu as pltpu
from jax.experimental.pallas import tpu_sc as plsc

N = 10000
E = 160000
T = 12
C = 32
D = T * C
NQ = 2
HQ = D // NQ
NP = 10240
NC = 2
NS = 16
KE = 128
NCH = 80
EPW_S = KE * NCH
EP_S = NS * EPW_S
EPW_H = EP_S // 32
JUNK = N + 16
NBUF = 4


def _deg_body(ei_hbm, dego_hbm, degi_hbm,
              ebuf, hist_o, hist_i, rbuf, sh_o, sh_i):
    cid = lax.axis_index("c")
    sid = lax.axis_index("s")
    w = sid * NC + cid
    zero16 = jnp.zeros((16,), jnp.float32)
    one16 = jnp.ones((16,), jnp.float32)
    lane = lax.broadcasted_iota(jnp.int32, (16,), 0)

    def zero_hist(i, _):
        hist_o[pl.ds(i * 16, 16)] = zero16
        hist_i[pl.ds(i * 16, 16)] = zero16
        return 0
    lax.fori_loop(0, NP // 16, zero_hist, 0)

    def do_hist(base, hist):
        pltpu.sync_copy(ei_hbm.at[pl.ds(base + w * EPW_H, EPW_H)], ebuf)

        def acc(j, _):
            ids = ebuf[pl.ds(j * 16, 16)]
            ids = jnp.where(lane + (w * EPW_H + j * 16) >= E, JUNK, ids)
            plsc.addupdate_scatter(hist, [ids], one16)
            return 0
        lax.fori_loop(0, EPW_H // 16, acc, 0)

    do_hist(0, hist_o)
    do_hist(EP_S, hist_i)

    pltpu.sync_copy(hist_o, sh_o.at[sid])
    pltpu.sync_copy(hist_i, sh_i.at[sid])
    plsc.subcore_barrier()

    cols = NP // NS

    def reduce_one(sh, hist, out_hbm):
        pltpu.sync_copy(sh.at[:, pl.ds(sid * cols, cols)], rbuf)

        def red(i, _):
            s = rbuf[0, pl.ds(i * 16, 16)]
            for k in range(1, NS):
                s = s + rbuf[k, pl.ds(i * 16, 16)]
            hist[pl.ds(i * 16, 16)] = s
            return 0
        lax.fori_loop(0, cols // 16, red, 0)
        pltpu.sync_copy(hist.at[pl.ds(0, cols)],
                        out_hbm.at[pl.ds(cid * NP + sid * cols, cols)])

    reduce_one(sh_o, hist_o, dego_hbm)
    reduce_one(sh_i, hist_i, degi_hbm)


def _scat_body(y_hbm, ei_hbm, zeros_hbm, out_hbm,
               sidx, didx, r0, r1, r2, r3,
               g0, g1, g2, g3, s0, s1, s2, s3, acc):
    cid = lax.axis_index("c")
    sid = lax.axis_index("s")
    rows = NP // NS
    rbufs = (r0, r1, r2, r3)
    gsems = (g0, g1, g2, g3)
    ssems = (s0, s1, s2, s3)
    lane = lax.broadcasted_iota(jnp.int32, (16,), 0)

    pltpu.sync_copy(ei_hbm.at[pl.ds(sid * NCH, NCH)], sidx)
    pltpu.sync_copy(ei_hbm.at[pl.ds((NS + sid) * NCH, NCH)], didx)

    @pl.when(sid == NS - 1)
    def _():
        def fix(r, _):
            for i in range(KE // 16):
                e0 = sid * EPW_S + r * KE + i * 16
                v = didx[r, pl.ds(i * 16, 16)]
                didx[r, pl.ds(i * 16, 16)] = jnp.where(
                    lane + e0 >= E, JUNK, v)
            return 0
        lax.fori_loop(0, NCH, fix, 0)

    for p in range(NQ // NC):
        q = cid * (NQ // NC) + p

        def mk(r, _):
            for i in range(KE // 16):
                s16 = sidx[r, pl.ds(i * 16, 16)]
                sidx[r, pl.ds(i * 16, 16)] = (
                    s16 * NQ + q if p == 0 else s16 + 1)
            return 0
        lax.fori_loop(0, NCH, mk, 0)

        pltpu.sync_copy(zeros_hbm, r0)
        for k in range(rows // KE):
            pltpu.sync_copy(r0, acc.at[pl.ds(sid * rows + k * KE, KE)])
        plsc.subcore_barrier()

        for b in range(NBUF):
            pltpu.async_copy(y_hbm.at[sidx.at[b]], rbufs[b], gsems[b])

        def body(jj, _):
            for b in range(NBUF):
                j = jj * NBUF + b
                rb, gs, ss = rbufs[b], gsems[b], ssems[b]
                pltpu.make_async_copy(y_hbm.at[sidx.at[j]], rb, gs).wait()
                pltpu.async_copy(rb, acc.at[didx.at[j]], ss, add=True)

                @pl.when(j < NCH - NBUF)
                def _():
                    pltpu.make_async_copy(rb, acc.at[didx.at[j]], ss).wait()
                    pltpu.async_copy(y_hbm.at[sidx.at[j + NBUF]], rb, gs)
            return 0
        lax.fori_loop(0, NCH // NBUF, body, 0)

        for b in range(NBUF):
            j = NCH - NBUF + b
            pltpu.make_async_copy(rbufs[b], acc.at[didx.at[j]],
                                  ssems[b]).wait()
        plsc.subcore_barrier()

        pltpu.sync_copy(
            acc.at[pl.ds(sid * rows, rows)],
            out_hbm.at[pl.ds(sid * rows, rows), pl.ds(q * HQ, HQ)])
        plsc.subcore_barrier()


@functools.lru_cache(maxsize=None)
def _sc_kernels():
    mesh = plsc.VectorSubcoreMesh(
        core_axis_name="c", subcore_axis_name="s",
        num_cores=NC, num_subcores=NS)
    params = pltpu.CompilerParams(needs_layout_passes=False,
                                  use_tc_tiling_on_sc=False)
    deg_kernel = functools.partial(
        pl.kernel,
        out_type=(jax.ShapeDtypeStruct((NC * NP,), jnp.float32),
                  jax.ShapeDtypeStruct((NC * NP,), jnp.float32)),
        mesh=mesh,
        compiler_params=params,
        scratch_types=[
            pltpu.VMEM((EPW_H,), jnp.int32),
            pltpu.VMEM((NP,), jnp.float32),
            pltpu.VMEM((NP,), jnp.float32),
            pltpu.VMEM((NS, NP // NS), jnp.float32),
            pltpu.VMEM_SHARED((NS, NP), jnp.float32),
            pltpu.VMEM_SHARED((NS, NP), jnp.float32),
        ],
    )(_deg_body)
    scat_kernel = functools.partial(
        pl.kernel,
        out_type=jax.ShapeDtypeStruct((NP, D), jnp.bfloat16),
        mesh=mesh,
        compiler_params=params,
        scratch_types=(
            [pltpu.VMEM((NCH, KE), jnp.int32)] * 2
            + [pltpu.VMEM((KE, HQ), jnp.bfloat16)] * NBUF
            + [pltpu.SemaphoreType.DMA] * (2 * NBUF)
            + [pltpu.VMEM_SHARED((NP, HQ), jnp.bfloat16)]),
    )(_scat_body)
    return deg_kernel, scat_kernel


def _prep_body(x_ref, d_ref, w_ref, o_ref):
    deg = jnp.sum(d_ref[...], axis=1, keepdims=True)
    norm = lax.rsqrt(jnp.maximum(deg, 1.0))
    xb = x_ref[...]
    wm = w_ref[...]
    for t in range(T):
        yt = lax.dot_general(xb[:, t, :], wm, (((0,), (0,)), ((), ())),
                             preferred_element_type=jnp.float32)
        o_ref[:, t * C:(t + 1) * C] = (yt * norm).astype(jnp.bfloat16)


def _epi_body(a_ref, d_ref, b_ref, o_ref):
    deg = jnp.sum(d_ref[...], axis=0, keepdims=True)
    norm = lax.rsqrt(jnp.maximum(deg, 1.0))
    ab = a_ref[...]
    bb = b_ref[...].reshape(C, 1)
    eye = jnp.eye(C, dtype=jnp.bfloat16)
    for t in range(T):
        at = lax.dot_general(eye, ab[:, t * C:(t + 1) * C],
                             (((1,), (1,)), ((), ())),
                             preferred_element_type=jnp.float32)
        o_ref[:, t, :] = jnp.maximum(at * norm + bb, 0.0)


def kernel(x, edge_index, W, b):
    ei_p = jnp.pad(edge_index, ((0, 0), (0, EP_S - E)))

    deg_kernel, scat_kernel = _sc_kernels()
    dego_f, degi_f = deg_kernel(ei_p.reshape(2 * EP_S))
    dego = dego_f.reshape(NC, NP).T
    degi = degi_f.reshape(NC, NP)

    x0 = jnp.pad(x[0], ((0, 0), (0, 0), (0, NP - N)))

    grid = NP // 128
    y = pl.pallas_call(
        _prep_body,
        grid=(grid,),
        in_specs=[pl.BlockSpec((C, T, 128), lambda i: (0, 0, i)),
                  pl.BlockSpec((128, NC), lambda i: (i, 0)),
                  pl.BlockSpec((C, C), lambda i: (0, 0))],
        out_specs=pl.BlockSpec((128, D), lambda i: (i, 0)),
        out_shape=jax.ShapeDtypeStruct((NP, D), jnp.bfloat16),
    )(x0, dego, W)

    y_tab = y.reshape(NQ * NP, HQ)
    zeros = jnp.zeros((KE, HQ), jnp.bfloat16)
    agg = scat_kernel(y_tab, ei_p.reshape(2 * NS * NCH, KE), zeros)

    z = pl.pallas_call(
        _epi_body,
        grid=(grid,),
        in_specs=[pl.BlockSpec((128, D), lambda i: (i, 0)),
                  pl.BlockSpec((NC, 128), lambda i: (0, i)),
                  pl.BlockSpec((1, C), lambda i: (0, 0))],
        out_specs=pl.BlockSpec((C, T, 128), lambda i: (0, 0, i)),
        out_shape=jax.ShapeDtypeStruct((C, T, NP), jnp.float32),
    )(agg, degi, b[None])

    return z[None, :, :, :N]


# speedup vs baseline: 1.2144x; 1.0282x over previous
import functools

import jax
import jax.numpy as jnp
from jax import lax
from jax.experimental import pallas as pl
from jax.experimental.pallas import tpu as pltpu
from jax.experimental.pallas import tpu_sc as plsc

N = 10000
E = 160000
T = 12
C = 32
D = T * C
NQ = 2
HQ = D // NQ
NP = 10240
NC = 2
NS = 16
KE = 128
NCH = 80
EPW_S = KE * NCH
EP_S = NS * EPW_S
EPW_H = EP_S // 32
JUNK = N + 16
NBUF = 4


def _deg_body(ei_hbm, dego_hbm, degi_hbm,
              ebuf, hist_o, hist_i, rbuf, sh_o, sh_i):
    cid = lax.axis_index("c")
    sid = lax.axis_index("s")
    w = sid * NC + cid
    zero16 = jnp.zeros((16,), jnp.float32)
    one16 = jnp.ones((16,), jnp.float32)
    lane = lax.broadcasted_iota(jnp.int32, (16,), 0)

    def zero_hist(i, _):
        hist_o[pl.ds(i * 16, 16)] = zero16
        hist_i[pl.ds(i * 16, 16)] = zero16
        return 0
    lax.fori_loop(0, NP // 16, zero_hist, 0)

    def do_hist(base, hist):
        pltpu.sync_copy(ei_hbm.at[pl.ds(base + w * EPW_H, EPW_H)], ebuf)

        def acc(j, _):
            ids = ebuf[pl.ds(j * 16, 16)]
            ids = jnp.where(lane + (w * EPW_H + j * 16) >= E, JUNK, ids)
            plsc.addupdate_scatter(hist, [ids], one16)
            return 0
        lax.fori_loop(0, EPW_H // 16, acc, 0)

    do_hist(0, hist_o)
    do_hist(EP_S, hist_i)

    pltpu.sync_copy(hist_o, sh_o.at[sid])
    pltpu.sync_copy(hist_i, sh_i.at[sid])
    plsc.subcore_barrier()

    cols = NP // NS

    def reduce_one(sh, hist, out_hbm):
        pltpu.sync_copy(sh.at[:, pl.ds(sid * cols, cols)], rbuf)

        def red(i, _):
            s = rbuf[0, pl.ds(i * 16, 16)]
            for k in range(1, NS):
                s = s + rbuf[k, pl.ds(i * 16, 16)]
            hist[pl.ds(i * 16, 16)] = s
            return 0
        lax.fori_loop(0, cols // 16, red, 0)
        pltpu.sync_copy(hist.at[pl.ds(0, cols)],
                        out_hbm.at[pl.ds(cid * NP + sid * cols, cols)])

    reduce_one(sh_o, hist_o, dego_hbm)
    reduce_one(sh_i, hist_i, degi_hbm)


def _scat_body(y_hbm, ei_hbm, zeros_hbm, out_hbm,
               sidx, didx, r0, r1, r2, r3,
               g0, g1, g2, g3, s0, s1, s2, s3, acc):
    cid = lax.axis_index("c")
    sid = lax.axis_index("s")
    rows = NP // NS
    rbufs = (r0, r1, r2, r3)
    gsems = (g0, g1, g2, g3)
    ssems = (s0, s1, s2, s3)
    lane = lax.broadcasted_iota(jnp.int32, (16,), 0)

    pltpu.sync_copy(ei_hbm.at[pl.ds(sid * NCH, NCH)], sidx)
    pltpu.sync_copy(ei_hbm.at[pl.ds((NS + sid) * NCH, NCH)], didx)

    @pl.when(sid == NS - 1)
    def _():
        def fix(r, _):
            for i in range(KE // 16):
                e0 = sid * EPW_S + r * KE + i * 16
                v = didx[r, pl.ds(i * 16, 16)]
                didx[r, pl.ds(i * 16, 16)] = jnp.where(
                    lane + e0 >= E, JUNK, v)
            return 0
        lax.fori_loop(0, NCH, fix, 0)

    for p in range(NQ // NC):
        q = cid * (NQ // NC) + p

        def mk(r, _):
            for i in range(KE // 16):
                s16 = sidx[r, pl.ds(i * 16, 16)]
                sidx[r, pl.ds(i * 16, 16)] = (
                    s16 * NQ + q if p == 0 else s16 + 1)
            return 0
        lax.fori_loop(0, NCH, mk, 0)

        pltpu.sync_copy(zeros_hbm, r0)
        for k in range(rows // KE):
            pltpu.sync_copy(r0, acc.at[pl.ds(sid * rows + k * KE, KE)])
        plsc.subcore_barrier()

        for b in range(NBUF):
            pltpu.async_copy(y_hbm.at[sidx.at[b]], rbufs[b], gsems[b])

        def body(jj, _):
            for b in range(NBUF):
                j = jj * NBUF + b
                rb, gs, ss = rbufs[b], gsems[b], ssems[b]
                pltpu.make_async_copy(y_hbm.at[sidx.at[j]], rb, gs).wait()
                pltpu.async_copy(rb, acc.at[didx.at[j]], ss, add=True)

                @pl.when(j < NCH - NBUF)
                def _():
                    pltpu.make_async_copy(rb, acc.at[didx.at[j]], ss).wait()
                    pltpu.async_copy(y_hbm.at[sidx.at[j + NBUF]], rb, gs)
            return 0
        lax.fori_loop(0, NCH // NBUF, body, 0)

        for b in range(NBUF):
            j = NCH - NBUF + b
            pltpu.make_async_copy(rbufs[b], acc.at[didx.at[j]],
                                  ssems[b]).wait()
        plsc.subcore_barrier()

        pltpu.sync_copy(
            acc.at[pl.ds(sid * rows, rows)],
            out_hbm.at[pl.ds(sid * rows, rows), pl.ds(q * HQ, HQ)])
        plsc.subcore_barrier()


@functools.lru_cache(maxsize=None)
def _sc_kernels():
    mesh = plsc.VectorSubcoreMesh(
        core_axis_name="c", subcore_axis_name="s",
        num_cores=NC, num_subcores=NS)
    params = pltpu.CompilerParams(needs_layout_passes=False,
                                  use_tc_tiling_on_sc=False)
    deg_kernel = functools.partial(
        pl.kernel,
        out_type=(jax.ShapeDtypeStruct((NC * NP,), jnp.float32),
                  jax.ShapeDtypeStruct((NC * NP,), jnp.float32)),
        mesh=mesh,
        compiler_params=params,
        scratch_types=[
            pltpu.VMEM((EPW_H,), jnp.int32),
            pltpu.VMEM((NP,), jnp.float32),
            pltpu.VMEM((NP,), jnp.float32),
            pltpu.VMEM((NS, NP // NS), jnp.float32),
            pltpu.VMEM_SHARED((NS, NP), jnp.float32),
            pltpu.VMEM_SHARED((NS, NP), jnp.float32),
        ],
    )(_deg_body)
    scat_kernel = functools.partial(
        pl.kernel,
        out_type=jax.ShapeDtypeStruct((NP, D), jnp.bfloat16),
        mesh=mesh,
        compiler_params=params,
        scratch_types=(
            [pltpu.VMEM((NCH, KE), jnp.int32)] * 2
            + [pltpu.VMEM((KE, HQ), jnp.bfloat16)] * NBUF
            + [pltpu.SemaphoreType.DMA] * (2 * NBUF)
            + [pltpu.VMEM_SHARED((NP, HQ), jnp.bfloat16)]),
    )(_scat_body)
    return deg_kernel, scat_kernel


def _prep_body(x_ref, d_ref, w_ref, o_ref):
    deg = jnp.sum(d_ref[...], axis=1, keepdims=True)
    norm = lax.rsqrt(jnp.maximum(deg, 1.0))
    xb = x_ref[...]
    wm = w_ref[...]
    for t in range(T):
        yt = lax.dot_general(xb[:, t, :], wm, (((0,), (0,)), ((), ())),
                             preferred_element_type=jnp.float32)
        o_ref[:, t * C:(t + 1) * C] = (yt * norm).astype(jnp.bfloat16)


def _epi_body(a_ref, d_ref, b_ref, o_ref):
    deg = jnp.sum(d_ref[...], axis=0, keepdims=True)
    norm = lax.rsqrt(jnp.maximum(deg, 1.0))
    ab = a_ref[...]
    bb = b_ref[...].reshape(C, 1)
    eye = jnp.eye(C, dtype=jnp.bfloat16)
    for t in range(T):
        at = lax.dot_general(eye, ab[:, t * C:(t + 1) * C],
                             (((1,), (1,)), ((), ())),
                             preferred_element_type=jnp.float32)
        o_ref[:, t, :] = jnp.maximum(at * norm + bb, 0.0)


def kernel(x, edge_index, W, b):
    ei_p = jnp.pad(edge_index, ((0, 0), (0, EP_S - E)))

    deg_kernel, scat_kernel = _sc_kernels()
    dego_f, degi_f = deg_kernel(ei_p.reshape(2 * EP_S))
    dego = dego_f.reshape(NC, NP).T
    degi = degi_f.reshape(NC, NP)

    grid = (N + 127) // 128
    NPY = grid * 128
    y = pl.pallas_call(
        _prep_body,
        grid=(grid,),
        in_specs=[pl.BlockSpec((C, T, 128), lambda i: (0, 0, i)),
                  pl.BlockSpec((128, NC), lambda i: (i, 0)),
                  pl.BlockSpec((C, C), lambda i: (0, 0))],
        out_specs=pl.BlockSpec((128, D), lambda i: (i, 0)),
        out_shape=jax.ShapeDtypeStruct((NPY, D), jnp.bfloat16),
    )(x[0], dego, W)

    y_tab = y.reshape(NQ * NPY, HQ)
    zeros = jnp.zeros((KE, HQ), jnp.bfloat16)
    agg = scat_kernel(y_tab, ei_p.reshape(2 * NS * NCH, KE), zeros)

    z = pl.pallas_call(
        _epi_body,
        grid=(grid,),
        in_specs=[pl.BlockSpec((128, D), lambda i: (i, 0)),
                  pl.BlockSpec((NC, 128), lambda i: (0, i)),
                  pl.BlockSpec((1, C), lambda i: (0, 0))],
        out_specs=pl.BlockSpec((C, T, 128), lambda i: (0, 0, i)),
        out_shape=jax.ShapeDtypeStruct((C, T, N), jnp.float32),
    )(agg, degi, b[None])

    return z[None]


# speedup vs baseline: 1.2861x; 1.0590x over previous
import functools

import jax
import jax.numpy as jnp
from jax import lax
from jax.experimental import pallas as pl
from jax.experimental.pallas import tpu as pltpu
from jax.experimental.pallas import tpu_sc as plsc

N = 10000
E = 160000
T = 12
C = 32
D = T * C
NQ = 2
HQ = D // NQ
NP = 10240
NC = 2
NS = 16
KE = 128
NCH = 80
EPW_S = KE * NCH
EP_S = NS * EPW_S
EPW_H = EP_S // 32
JUNK = N + 16
NBUF = 4


def _deg_body(ei_hbm, dego_hbm, degi_hbm,
              ebuf, hist_o, hist_i, rbuf, sh_o, sh_i):
    cid = lax.axis_index("c")
    sid = lax.axis_index("s")
    w = sid * NC + cid
    zero16 = jnp.zeros((16,), jnp.float32)
    one16 = jnp.ones((16,), jnp.float32)
    lane = lax.broadcasted_iota(jnp.int32, (16,), 0)

    def zero_hist(i, _):
        hist_o[pl.ds(i * 16, 16)] = zero16
        hist_i[pl.ds(i * 16, 16)] = zero16
        return 0
    lax.fori_loop(0, NP // 16, zero_hist, 0)

    def do_hist(base, hist):
        pltpu.sync_copy(ei_hbm.at[pl.ds(base + w * EPW_H, EPW_H)], ebuf)

        def acc(j, _):
            ids = ebuf[pl.ds(j * 16, 16)]
            ids = jnp.where(lane + (w * EPW_H + j * 16) >= E, JUNK, ids)
            plsc.addupdate_scatter(hist, [ids], one16)
            return 0
        lax.fori_loop(0, EPW_H // 16, acc, 0)

    do_hist(0, hist_o)
    do_hist(EP_S, hist_i)

    pltpu.sync_copy(hist_o, sh_o.at[sid])
    pltpu.sync_copy(hist_i, sh_i.at[sid])
    plsc.subcore_barrier()

    cols = NP // NS

    def reduce_one(sh, hist, out_hbm):
        pltpu.sync_copy(sh.at[:, pl.ds(sid * cols, cols)], rbuf)

        def red(i, _):
            s = rbuf[0, pl.ds(i * 16, 16)]
            for k in range(1, NS):
                s = s + rbuf[k, pl.ds(i * 16, 16)]
            hist[pl.ds(i * 16, 16)] = s
            return 0
        lax.fori_loop(0, cols // 16, red, 0)
        pltpu.sync_copy(hist.at[pl.ds(0, cols)],
                        out_hbm.at[pl.ds(cid * NP + sid * cols, cols)])

    reduce_one(sh_o, hist_o, dego_hbm)
    reduce_one(sh_i, hist_i, degi_hbm)


def _scat_body(y_hbm, ei_hbm, zeros_hbm, out_hbm,
               sidx, didx, r0, r1, r2, r3,
               g0, g1, g2, g3, s0, s1, s2, s3, acc):
    cid = lax.axis_index("c")
    sid = lax.axis_index("s")
    rows = NP // NS
    rbufs = (r0, r1, r2, r3)
    gsems = (g0, g1, g2, g3)
    ssems = (s0, s1, s2, s3)
    lane = lax.broadcasted_iota(jnp.int32, (16,), 0)

    pltpu.sync_copy(ei_hbm.at[pl.ds(sid * NCH, NCH)], sidx)
    pltpu.sync_copy(ei_hbm.at[pl.ds((NS + sid) * NCH, NCH)], didx)

    @pl.when(sid == NS - 1)
    def _():
        def fix(r, _):
            for i in range(KE // 16):
                e0 = sid * EPW_S + r * KE + i * 16
                v = didx[r, pl.ds(i * 16, 16)]
                didx[r, pl.ds(i * 16, 16)] = jnp.where(
                    lane + e0 >= E, JUNK, v)
            return 0
        lax.fori_loop(0, NCH, fix, 0)

    for p in range(NQ // NC):
        q = cid * (NQ // NC) + p

        def mk(r, _):
            for i in range(KE // 16):
                s16 = sidx[r, pl.ds(i * 16, 16)]
                sidx[r, pl.ds(i * 16, 16)] = (
                    s16 * NQ + q if p == 0 else s16 + 1)
            return 0
        lax.fori_loop(0, NCH, mk, 0)

        pltpu.sync_copy(zeros_hbm, r0)
        for k in range(rows // KE):
            pltpu.sync_copy(r0, acc.at[pl.ds(sid * rows + k * KE, KE)])
        plsc.subcore_barrier()

        for b in range(NBUF):
            pltpu.async_copy(y_hbm.at[sidx.at[b]], rbufs[b], gsems[b])

        def body(jj, _):
            for b in range(NBUF):
                j = jj * NBUF + b
                rb, gs, ss = rbufs[b], gsems[b], ssems[b]
                pltpu.make_async_copy(y_hbm.at[sidx.at[j]], rb, gs).wait()
                pltpu.async_copy(rb, acc.at[didx.at[j]], ss, add=True)

                @pl.when(j < NCH - NBUF)
                def _():
                    pltpu.make_async_copy(rb, acc.at[didx.at[j]], ss).wait()
                    pltpu.async_copy(y_hbm.at[sidx.at[j + NBUF]], rb, gs)
            return 0
        lax.fori_loop(0, NCH // NBUF, body, 0)

        for b in range(NBUF):
            j = NCH - NBUF + b
            pltpu.make_async_copy(rbufs[b], acc.at[didx.at[j]],
                                  ssems[b]).wait()
        plsc.subcore_barrier()

        pltpu.sync_copy(
            acc.at[pl.ds(sid * rows, rows)],
            out_hbm.at[pl.ds(sid * rows, rows), pl.ds(q * HQ, HQ)])
        plsc.subcore_barrier()


@functools.lru_cache(maxsize=None)
def _sc_kernels():
    mesh = plsc.VectorSubcoreMesh(
        core_axis_name="c", subcore_axis_name="s",
        num_cores=NC, num_subcores=NS)
    params = pltpu.CompilerParams(needs_layout_passes=False,
                                  use_tc_tiling_on_sc=False)
    deg_kernel = functools.partial(
        pl.kernel,
        out_type=(jax.ShapeDtypeStruct((NC * NP,), jnp.float32),
                  jax.ShapeDtypeStruct((NC * NP,), jnp.float32)),
        mesh=mesh,
        compiler_params=params,
        scratch_types=[
            pltpu.VMEM((EPW_H,), jnp.int32),
            pltpu.VMEM((NP,), jnp.float32),
            pltpu.VMEM((NP,), jnp.float32),
            pltpu.VMEM((NS, NP // NS), jnp.float32),
            pltpu.VMEM_SHARED((NS, NP), jnp.float32),
            pltpu.VMEM_SHARED((NS, NP), jnp.float32),
        ],
    )(_deg_body)
    scat_kernel = functools.partial(
        pl.kernel,
        out_type=jax.ShapeDtypeStruct((NP, D), jnp.bfloat16),
        mesh=mesh,
        compiler_params=params,
        scratch_types=(
            [pltpu.VMEM((NCH, KE), jnp.int32)] * 2
            + [pltpu.VMEM((KE, HQ), jnp.bfloat16)] * NBUF
            + [pltpu.SemaphoreType.DMA] * (2 * NBUF)
            + [pltpu.VMEM_SHARED((NP, HQ), jnp.bfloat16)]),
    )(_scat_body)
    return deg_kernel, scat_kernel


def _prep_body(x_ref, w_ref, o_ref):
    xb = x_ref[...]
    wm = w_ref[...]
    for t in range(T):
        yt = lax.dot_general(xb[:, t, :], wm, (((0,), (0,)), ((), ())),
                             preferred_element_type=jnp.float32)
        o_ref[:, t * C:(t + 1) * C] = yt.astype(jnp.bfloat16)


def _epi_body(a_ref, d_ref, b_ref, o_ref):
    deg = jnp.sum(d_ref[...], axis=0, keepdims=True)
    norm = lax.rsqrt(jnp.maximum(deg, 1.0))
    ab = a_ref[...]
    bb = b_ref[...].reshape(C, 1)
    eye = jnp.eye(C, dtype=jnp.bfloat16)
    for t in range(T):
        at = lax.dot_general(eye, ab[:, t * C:(t + 1) * C],
                             (((1,), (1,)), ((), ())),
                             preferred_element_type=jnp.float32)
        o_ref[:, t, :] = jnp.maximum(at * norm + bb, 0.0)


def kernel(x, edge_index, W, b):
    ei_p = jnp.pad(edge_index, ((0, 0), (0, EP_S - E)))

    deg_kernel, scat_kernel = _sc_kernels()
    dego_f, degi_f = deg_kernel(ei_p.reshape(2 * EP_S))
    degi = degi_f.reshape(NC, NP)

    grid = (N + 127) // 128
    NPY = grid * 128
    y0 = pl.pallas_call(
        _prep_body,
        grid=(grid,),
        in_specs=[pl.BlockSpec((C, T, 128), lambda i: (0, 0, i)),
                  pl.BlockSpec((C, C), lambda i: (0, 0))],
        out_specs=pl.BlockSpec((128, D), lambda i: (i, 0)),
        out_shape=jax.ShapeDtypeStruct((NPY, D), jnp.bfloat16),
    )(x[0], W)

    dego = dego_f.reshape(NC, NP)
    ns = lax.rsqrt(jnp.maximum(dego[0, :NPY] + dego[1, :NPY], 1.0))
    y = y0 * ns[:, None].astype(jnp.bfloat16)

    y_tab = y.reshape(NQ * NPY, HQ)
    zeros = jnp.zeros((KE, HQ), jnp.bfloat16)
    agg = scat_kernel(y_tab, ei_p.reshape(2 * NS * NCH, KE), zeros)

    z = pl.pallas_call(
        _epi_body,
        grid=(grid,),
        in_specs=[pl.BlockSpec((128, D), lambda i: (i, 0)),
                  pl.BlockSpec((NC, 128), lambda i: (0, i)),
                  pl.BlockSpec((1, C), lambda i: (0, 0))],
        out_specs=pl.BlockSpec((C, T, 128), lambda i: (0, 0, i)),
        out_shape=jax.ShapeDtypeStruct((C, T, N), jnp.float32),
    )(agg, degi, b[None])

    return z[None]
